# baseline (speedup 1.0000x reference)
import functools

import jax
import jax.numpy as jnp
from jax import lax
from jax.experimental import pallas as pl
from jax.experimental.pallas import tpu as pltpu
from jax.experimental.pallas import tpu_sc as plsc

VOCAB = 1000000
EMBED_DIM = 32
BATCH = 4096
SEQ = 200

BB = 128
PLANE = BATCH * SEQ
UNROLL = 8


def _embed_body(xt_hbm, table_hbm, out_hbm, idx_v, rows0, rows1, tr0, tr1,
                gsem0, gsem1, osem0, osem1):
    info = plsc.get_sparse_core_info()
    nc = info.num_cores

    w = lax.axis_index("s") * nc + lax.axis_index("c")

    pltpu.sync_copy(xt_hbm.at[:, pl.ds(w * BB, BB)], idx_v)

    rows = (rows0, rows1)
    tr = (tr0, tr1)
    gsem = (gsem0, gsem1)
    osem = (osem0, osem1)

    lane = lax.iota(jnp.int32, 16)
    c_lo = lane
    c_hi = lane + 16

    def start_gather(l, k):
        pltpu.make_async_copy(
            table_hbm.at[idx_v.at[l]], rows[k], gsem[k]).start()

    def wait_gather(k):
        pltpu.make_async_copy(
            table_hbm.at[idx_v.at[0]], rows[k], gsem[k]).wait()

    def line_off(l):
        return (l // 8) * (32 * 1024) + w * 1024 + (l % 8) * 128

    def start_out(l, k):
        pltpu.make_async_copy(
            tr[k], out_hbm.at[:, pl.ds(line_off(l), BB)], osem[k]).start()

    def wait_out(k):
        pltpu.make_async_copy(
            tr[k], out_hbm.at[:, pl.ds(0, BB)], osem[k]).wait()

    def transpose(k):
        rv, tv = rows[k], tr[k]

        def t_step(i, c2):
            for dj in range(UNROLL):
                b = i * UNROLL + dj
                b_vec = jnp.full((16,), b, jnp.int32)
                v0 = rv[b, pl.ds(0, 16)]
                v1 = rv[b, pl.ds(16, 16)]
                plsc.store_scatter(tv, [c_lo, b_vec], v0)
                plsc.store_scatter(tv, [c_hi, b_vec], v1)
            return c2

        lax.fori_loop(0, BB // UNROLL, t_step, 0)

    start_gather(0, 0)

    def pair(li, carry):
        l0 = li * 2

        start_gather(l0 + 1, 1)
        wait_gather(0)

        @pl.when(li > 0)
        def _():
            wait_out(0)

        transpose(0)
        start_out(l0, 0)

        @pl.when(l0 + 2 < SEQ)
        def _():
            start_gather(l0 + 2, 0)

        wait_gather(1)

        @pl.when(li > 0)
        def _():
            wait_out(1)

        transpose(1)
        start_out(l0 + 1, 1)
        return carry

    lax.fori_loop(0, SEQ // 2, pair, 0)
    wait_out(0)
    wait_out(1)


def kernel(x, table):
    mesh = plsc.VectorSubcoreMesh(core_axis_name="c", subcore_axis_name="s")

    f = functools.partial(
        pl.kernel,
        mesh=mesh,
        compiler_params=pltpu.CompilerParams(
            use_tc_tiling_on_sc=False, needs_layout_passes=False),
        out_type=jax.ShapeDtypeStruct((EMBED_DIM, PLANE), jnp.float32),
        scratch_types=[
            pltpu.VMEM((SEQ, BB), jnp.int32),
            pltpu.VMEM((BB, EMBED_DIM), jnp.float32),
            pltpu.VMEM((BB, EMBED_DIM), jnp.float32),
            pltpu.VMEM((EMBED_DIM, BB), jnp.float32),
            pltpu.VMEM((EMBED_DIM, BB), jnp.float32),
            pltpu.SemaphoreType.DMA,
            pltpu.SemaphoreType.DMA,
            pltpu.SemaphoreType.DMA,
            pltpu.SemaphoreType.DMA,
        ],
    )(_embed_body)
    out2 = f(jnp.transpose(x), table)
    v = out2.reshape(EMBED_DIM, SEQ // 8, BATCH // BB, 8, BB)
    return v.transpose(2, 4, 0, 1, 3).reshape(BATCH, EMBED_DIM, SEQ)
